# baseline (speedup 1.0000x reference)
import functools

import jax
import jax.numpy as jnp
from jax import lax
from jax.experimental import pallas as pl
from jax.experimental.pallas import tpu as pltpu
from jax.experimental.pallas import tpu_sc as plsc

VOCAB = 100000
D = 64
BATCH = 4096
TEXT_LEN = 200
TOPIC_LEN = 20

NC = 2
NS = 16
NW = NC * NS

CHUNK = 128
K1 = 2
K2 = 1

B1 = BATCH * TEXT_LEN
B2 = BATCH * TOPIC_LEN
Q1 = B1 // 2
Q2 = B2 // 2
PW1 = Q1 // NW
PW2 = Q2 // NW
NCH1 = PW1 // CHUNK
NCH2 = PW2 // CHUNK


def _make_kernel():
    mesh = plsc.VectorSubcoreMesh(core_axis_name="c", subcore_axis_name="s")

    @functools.partial(
        pl.kernel,
        mesh=mesh,
        compiler_params=pltpu.CompilerParams(use_tc_tiling_on_sc=False),
        out_type=(
            jax.ShapeDtypeStruct((Q1, 2 * D), jnp.float32),
            jax.ShapeDtypeStruct((Q2, 2 * D), jnp.float32),
        ),
        scratch_types=[
            pltpu.VMEM((2, NCH1, CHUNK), jnp.int32),
            pltpu.VMEM((2, NCH2, CHUNK), jnp.int32),
            pltpu.VMEM((2, 2, K1 * CHUNK, D), jnp.float32),
            [pltpu.SemaphoreType.DMA] * 2,
            [pltpu.SemaphoreType.DMA] * 2,
        ],
    )
    def k(table, text_eo, topic_eo, out1, out2, idx1, idx2, rows, gsem, osem):
        wid = lax.axis_index("s") * NC + lax.axis_index("c")

        pltpu.sync_copy(text_eo.at[wid], idx1)
        pltpu.sync_copy(topic_eo.at[wid], idx2)

        def run(idx, nch, out, base, kk):
            ngroups = nch // kk

            def fire_g(t, s):
                for i in range(kk):
                    for e in range(2):
                        pltpu.async_copy(
                            table.at[idx.at[e, t * kk + i]],
                            rows.at[s, e, pl.ds(i * CHUNK, CHUNK)],
                            gsem[s],
                        )

            def drain_g(t, s):
                for i in range(kk):
                    for e in range(2):
                        pltpu.make_async_copy(
                            table.at[idx.at[e, t * kk + i]],
                            rows.at[s, e, pl.ds(i * CHUNK, CHUNK)],
                            gsem[s],
                        ).wait()

            def wb(t, s, e):
                return pltpu.make_async_copy(
                    rows.at[s, e, pl.ds(0, kk * CHUNK)],
                    out.at[
                        pl.ds(base + t * kk * CHUNK, kk * CHUNK),
                        pl.ds(e * D, D),
                    ],
                    osem[s],
                )

            def wb_start(t, s):
                wb(t, s, 0).start()
                wb(t, s, 1).start()

            def wb_wait(t, s):
                wb(t, s, 0).wait()
                wb(t, s, 1).wait()

            fire_g(0, 0)
            fire_g(1, 1)
            drain_g(0, 0)
            wb_start(0, 0)

            def body(p, _):
                t0 = 2 + 2 * p
                drain_g(t0 - 1, 1)
                wb_start(t0 - 1, 1)
                wb_wait(t0 - 2, 0)
                fire_g(t0, 0)
                drain_g(t0, 0)
                wb_start(t0, 0)
                wb_wait(t0 - 1, 1)
                fire_g(t0 + 1, 1)
                return 0

            lax.fori_loop(0, (ngroups - 2) // 2, body, 0)

            drain_g(ngroups - 1, 1)
            wb_start(ngroups - 1, 1)
            wb_wait(ngroups - 2, 0)
            wb_wait(ngroups - 1, 1)

        run(idx1, NCH1, out1, wid * PW1, K1)
        run(idx2, NCH2, out2, wid * PW2, K2)

    return k


_kern = _make_kernel()


def _eo(flat, nch):
    return jnp.stack(
        [flat[0::2].reshape(NW, nch, CHUNK), flat[1::2].reshape(NW, nch, CHUNK)],
        axis=1,
    )


def kernel(table, text, topic):
    text_eo = _eo(text.reshape(-1).astype(jnp.int32), NCH1)
    topic_eo = _eo(topic.reshape(-1).astype(jnp.int32), NCH2)
    out1, out2 = _kern(table, text_eo, topic_eo)
    return (
        out1.reshape(BATCH, TEXT_LEN, D),
        out2.reshape(BATCH, TOPIC_LEN, D),
    )


# speedup vs baseline: 1.1728x; 1.1728x over previous
import functools

import jax
import jax.numpy as jnp
from jax import lax
from jax.experimental import pallas as pl
from jax.experimental.pallas import tpu as pltpu
from jax.experimental.pallas import tpu_sc as plsc

VOCAB = 100000
D = 64
BATCH = 4096
TEXT_LEN = 200
TOPIC_LEN = 20

NC = 2
NS = 16
NW = NC * NS

BPW = BATCH // NW

HALF = TEXT_LEN // 2
NB1 = 2
NB2 = 8
NG1 = BPW // NB1
NG2 = BPW // NB2


def _make_kernel():
    mesh = plsc.VectorSubcoreMesh(core_axis_name="c", subcore_axis_name="s")

    @functools.partial(
        pl.kernel,
        mesh=mesh,
        compiler_params=pltpu.CompilerParams(use_tc_tiling_on_sc=False),
        out_type=(
            jax.ShapeDtypeStruct((BATCH, TEXT_LEN, D), jnp.float32),
            jax.ShapeDtypeStruct((BATCH, TOPIC_LEN, D), jnp.float32),
        ),
        scratch_types=[
            pltpu.VMEM((2 * BPW, HALF), jnp.int32),
            pltpu.VMEM((BPW, TOPIC_LEN), jnp.int32),
            pltpu.VMEM((2, NB1, TEXT_LEN, D), jnp.float32),
            pltpu.VMEM((2, NB2, TOPIC_LEN, D), jnp.float32),
            [pltpu.SemaphoreType.DMA] * 2,
            [pltpu.SemaphoreType.DMA] * 2,
        ],
    )
    def k(table, text, topic, out1, out2, idx1, idx2, rows1, rows2,
          gsem, osem):
        wid = lax.axis_index("s") * NC + lax.axis_index("c")

        pltpu.sync_copy(text.at[wid], idx1)
        pltpu.sync_copy(topic.at[wid], idx2)

        def run(gathers, rows, ngroups, out, nb):
            base = wid * BPW

            def fire_g(t, s):
                gathers(t, s, start=True)

            def drain_g(t, s):
                gathers(t, s, start=False)

            def wb(t, s):
                return pltpu.make_async_copy(
                    rows.at[s],
                    out.at[pl.ds(base + t * nb, nb)],
                    osem[s],
                )

            fire_g(0, 0)
            fire_g(1, 1)
            drain_g(0, 0)
            wb(0, 0).start()

            def body(p, _):
                t0 = 2 + 2 * p
                drain_g(t0 - 1, 1)
                wb(t0 - 1, 1).start()
                wb(t0 - 2, 0).wait()
                fire_g(t0, 0)
                drain_g(t0, 0)
                wb(t0, 0).start()
                wb(t0 - 1, 1).wait()
                fire_g(t0 + 1, 1)
                return 0

            lax.fori_loop(0, (ngroups - 2) // 2, body, 0)

            drain_g(ngroups - 1, 1)
            wb(ngroups - 1, 1).start()
            wb(ngroups - 2, 0).wait()
            wb(ngroups - 1, 1).wait()

        def gathers1(t, s, start):
            for bb in range(NB1):
                for h in range(2):
                    cp = pltpu.make_async_copy(
                        table.at[idx1.at[2 * (t * NB1 + bb) + h]],
                        rows1.at[s, bb, pl.ds(h * HALF, HALF)],
                        gsem[s],
                    )
                    cp.start() if start else cp.wait()

        def gathers2(t, s, start):
            for bb in range(NB2):
                cp = pltpu.make_async_copy(
                    table.at[idx2.at[t * NB2 + bb]],
                    rows2.at[s, bb],
                    gsem[s],
                )
                cp.start() if start else cp.wait()

        run(gathers1, rows1, NG1, out1, NB1)
        run(gathers2, rows2, NG2, out2, NB2)

    return k


_kern = _make_kernel()


def kernel(table, text, topic):
    text_r = text.astype(jnp.int32).reshape(NW, 2 * BPW, HALF)
    topic_r = topic.astype(jnp.int32).reshape(NW, BPW, TOPIC_LEN)
    return _kern(table, text_r, topic_r)


# speedup vs baseline: 1.2095x; 1.0313x over previous
import functools

import jax
import jax.numpy as jnp
from jax import lax
from jax.experimental import pallas as pl
from jax.experimental.pallas import tpu as pltpu
from jax.experimental.pallas import tpu_sc as plsc

VOCAB = 100000
D = 64
BATCH = 4096
TEXT_LEN = 200
TOPIC_LEN = 20

NC = 2
NS = 16
NW = NC * NS

CHUNK = 128
K1 = 4
K2 = 2

B1 = BATCH * TEXT_LEN
B2 = BATCH * TOPIC_LEN
PW1 = B1 // NW
PW2 = B2 // NW
NCH1 = PW1 // CHUNK
NCH2 = PW2 // CHUNK


def _make_kernel(nrows, nch, kk):
    mesh = plsc.VectorSubcoreMesh(core_axis_name="c", subcore_axis_name="s")
    pw = nrows // NW

    @functools.partial(
        pl.kernel,
        mesh=mesh,
        compiler_params=pltpu.CompilerParams(use_tc_tiling_on_sc=False),
        out_type=jax.ShapeDtypeStruct((nrows, D), jnp.float32),
        scratch_types=[
            pltpu.VMEM((nch, CHUNK), jnp.int32),
            pltpu.VMEM((2, kk * CHUNK, D), jnp.float32),
            [pltpu.SemaphoreType.DMA] * 2,
            [pltpu.SemaphoreType.DMA] * 2,
        ],
    )
    def k(table, indices, out_arr, idx_v, rows, gsem, osem):
        wid = lax.axis_index("s") * NC + lax.axis_index("c")

        pltpu.sync_copy(indices.at[wid], idx_v)

        def run(idx, nch, out, base, kk):
            ngroups = nch // kk

            def fire_g(t, s):
                for i in range(kk):
                    pltpu.async_copy(
                        table.at[idx.at[t * kk + i]],
                        rows.at[s, pl.ds(i * CHUNK, CHUNK)],
                        gsem[s],
                    )

            def drain_g(t, s):
                for i in range(kk):
                    pltpu.make_async_copy(
                        table.at[idx.at[t * kk + i]],
                        rows.at[s, pl.ds(i * CHUNK, CHUNK)],
                        gsem[s],
                    ).wait()

            def wb(t, s):
                return pltpu.make_async_copy(
                    rows.at[s, pl.ds(0, kk * CHUNK)],
                    out.at[pl.ds(base + t * kk * CHUNK, kk * CHUNK)],
                    osem[s],
                )

            fire_g(0, 0)
            fire_g(1, 1)
            drain_g(0, 0)
            wb(0, 0).start()

            def body(p, _):
                t0 = 2 + 2 * p
                drain_g(t0 - 1, 1)
                wb(t0 - 1, 1).start()
                wb(t0 - 2, 0).wait()
                fire_g(t0, 0)
                drain_g(t0, 0)
                wb(t0, 0).start()
                wb(t0 - 1, 1).wait()
                fire_g(t0 + 1, 1)
                return 0

            lax.fori_loop(0, (ngroups - 2) // 2, body, 0)

            drain_g(ngroups - 1, 1)
            wb(ngroups - 1, 1).start()
            wb(ngroups - 2, 0).wait()
            wb(ngroups - 1, 1).wait()

        run(idx_v, nch, out_arr, wid * pw, kk)

    return k


_kern_text = _make_kernel(B1, NCH1, K1)
_kern_topic = _make_kernel(B2, NCH2, K2)


def kernel(table, text, topic):
    text_r = text.reshape(NW, NCH1, CHUNK).astype(jnp.int32)
    topic_r = topic.reshape(NW, NCH2, CHUNK).astype(jnp.int32)
    out1 = _kern_text(table, text_r)
    out2 = _kern_topic(table, topic_r)
    return (
        out1.reshape(BATCH, TEXT_LEN, D),
        out2.reshape(BATCH, TOPIC_LEN, D),
    )
